# baseline (speedup 1.0000x reference)
import functools

import numpy as np

import jax
import jax.numpy as jnp
from jax import lax
from jax.experimental import pallas as pl
from jax.experimental.pallas import tpu as pltpu
from jax.experimental.pallas import tpu_sc as plsc

N = 10000
E = 320000
D = 128
ED = 16
NP = 10240
NC = 2
NS = 16
NW = NC * NS
EPW = E // NW
EB = 80
NB = EPW // EB
RPT = NP // NS
ZR = 16
F32 = jnp.float32

_mesh = plsc.VectorSubcoreMesh(core_axis_name="c", subcore_axis_name="s")

_PERM = np.array(
    [32 * (j // 32) + 16 * ((j % 32) % 2) + (j % 32) // 2 for j in range(D)],
    dtype=np.int32)


@functools.partial(
    pl.kernel,
    out_type=jax.ShapeDtypeStruct((NC, 1, NP), F32),
    mesh=_mesh,
    scratch_types=[
        pltpu.VMEM_SHARED((NP,), F32),
        pltpu.VMEM((NB, EB), jnp.int32),
        pltpu.VMEM((EB,), F32),
        pltpu.VMEM((RPT,), F32),
        pltpu.SemaphoreType.DMA,
    ],
)
def _deg_sc(col_hbm, deg_hbm, acc, colb, ones, zbuf, si):
    cid = lax.axis_index("c")
    sid = lax.axis_index("s")
    wid = sid * NC + cid
    pltpu.async_copy(col_hbm.at[wid], colb, si)
    for k in range(RPT // 16):
        zbuf[pl.ds(k * 16, 16)] = jnp.zeros((16,), F32)
    for k in range(EB // 16):
        ones[pl.ds(k * 16, 16)] = jnp.full((16,), 1.0, F32)
    pltpu.sync_copy(zbuf, acc.at[pl.ds(sid * RPT, RPT)])
    pltpu.make_async_copy(col_hbm.at[wid], colb, si).wait()
    plsc.subcore_barrier()

    def blk(i, _):
        pltpu.sync_copy(ones, acc.at[colb.at[i]], add=True)
        return 0

    lax.fori_loop(0, NB, blk, 0)
    plsc.subcore_barrier()
    s = pl.ds(sid * RPT, RPT)
    pltpu.sync_copy(acc.at[s], zbuf)
    pltpu.sync_copy(zbuf, deg_hbm.at[cid, 0, s])


@functools.partial(
    pl.kernel,
    out_type=jax.ShapeDtypeStruct((NC, NP, D), F32),
    mesh=_mesh,
    scratch_types=[
        pltpu.VMEM_SHARED((NP, D), F32),
        [pltpu.VMEM((EB,), jnp.int32)] * 2,
        [pltpu.VMEM((EB,), jnp.int32)] * 2,
        [pltpu.VMEM((EB, D), F32)] * 2,
        [pltpu.VMEM((EB, D // 2), jnp.int32)] * 2,
        pltpu.VMEM((ZR, D), F32),
        [pltpu.SemaphoreType.DMA] * 2,
        [pltpu.SemaphoreType.DMA] * 2,
        [pltpu.SemaphoreType.DMA] * 2,
    ],
)
def _mp_sc(h_hbm, m_hbm, row_hbm, col_hbm, out_hbm, acc, rowv, colv, rows, mv,
           zb, sg, sm, si):
    cid = lax.axis_index("c")
    sid = lax.axis_index("s")
    wid = sid * NC + cid
    base = wid * EPW

    def zrow(r, _):
        for c in range(D // 16):
            zb[r, pl.ds(c * 16, 16)] = jnp.zeros((16,), F32)
        return 0

    lax.fori_loop(0, ZR, zrow, 0)

    def zcopy(k, _):
        pltpu.sync_copy(zb, acc.at[pl.ds(sid * RPT + k * ZR, ZR)])
        return 0

    lax.fori_loop(0, RPT // ZR, zcopy, 0)
    plsc.subcore_barrier()

    def idx_copies(i, t):
        sl = pl.ds(base + i * EB, EB)
        return (pltpu.make_async_copy(row_hbm.at[sl], rowv[t], si[t]),
                pltpu.make_async_copy(col_hbm.at[sl], colv[t], si[t]))

    def g_copy(i, t):
        return pltpu.make_async_copy(h_hbm.at[rowv[t]], rows[t], sg[t])

    def m_copy(i, t):
        return pltpu.make_async_copy(m_hbm.at[wid * NB + i], mv[t], sm[t])

    for c in idx_copies(0, 0):
        c.start()
        c.wait()
    g_copy(0, 0).start()
    m_copy(0, 0).start()
    for c in idx_copies(1, 1):
        c.start()

    def slot(i, s, t):
        g_copy(i, s).wait()
        m_copy(i, s).wait()

        @pl.when(i + 1 < NB)
        def _():
            for c in idx_copies(i + 1, t):
                c.wait()
            g_copy(i + 1, t).start()
            m_copy(i + 1, t).start()

        def mul(b, _):
            for g in range(D // 32):
                w = mv[s][b, pl.ds(16 * g, 16)]
                lo = lax.bitcast_convert_type(w << 16, F32)
                hi = lax.bitcast_convert_type(w & jnp.int32(-65536), F32)
                sl = pl.ds(32 * g, 16)
                sh = pl.ds(32 * g + 16, 16)
                rows[s][b, sl] = rows[s][b, sl] * lo
                rows[s][b, sh] = rows[s][b, sh] * hi
            return 0

        lax.fori_loop(0, EB, mul, 0)
        pltpu.sync_copy(rows[s], acc.at[colv[s]], add=True)

        @pl.when(i + 2 < NB)
        def _():
            for c in idx_copies(i + 2, s):
                c.start()

    def ring(k, _):
        i0 = 2 * k
        slot(i0, 0, 1)

        @pl.when(i0 + 1 < NB)
        def _():
            slot(i0 + 1, 1, 0)

        return 0

    lax.fori_loop(0, (NB + 1) // 2, ring, 0)
    plsc.subcore_barrier()

    def wb(k, _):
        s = pl.ds(sid * RPT + k * ZR, ZR)
        pltpu.sync_copy(acc.at[s], zb)
        pltpu.sync_copy(zb, out_hbm.at[cid, s])
        return 0

    lax.fori_loop(0, RPT // ZR, wb, 0)


def _dis_of(deg_ref):
    deg = deg_ref[0, :] + deg_ref[1, :]
    return jnp.where(deg > 0, lax.rsqrt(deg), 0.0)


def _m_body(ea_ref, we_ref, be_ref, ew_ref, eb_ref, o_ref):
    ef = lax.dot_general(ea_ref[...], we_ref[...], (((1,), (0,)), ((), ())),
                         preferred_element_type=F32)
    ef = ef + be_ref[...][None, :]
    m = lax.dot_general(ef, ew_ref[...], (((1,), (1,)), ((), ())),
                        preferred_element_type=F32)
    o_ref[...] = (m + eb_ref[...][None, :]).astype(jnp.bfloat16)


def _h0_body(x_ref, lw_ref, lb_ref, deg_ref, o_ref):
    h = lax.dot_general(x_ref[...], lw_ref[...], (((1,), (1,)), ((), ())),
                        preferred_element_type=F32)
    h = h + lb_ref[...][None, :]
    o_ref[...] = h * _dis_of(deg_ref)[:, None]


def _hmid_body(p_ref, deg_ref, g_ref, b_ref, lw_ref, lb_ref, o_ref):
    dis = _dis_of(deg_ref)
    y = (p_ref[0] + p_ref[1]) * dis[:, None]
    y = jnp.maximum(y, 0.0)
    mu = jnp.mean(y, axis=-1, keepdims=True)
    var = jnp.mean((y - mu) ** 2, axis=-1, keepdims=True)
    z = (y - mu) / jnp.sqrt(var + 1e-5) * g_ref[...][None, :] + b_ref[...][None, :]
    h = lax.dot_general(z, lw_ref[...], (((1,), (1,)), ((), ())),
                        preferred_element_type=F32)
    h = h + lb_ref[...][None, :]
    o_ref[...] = h * dis[:, None]


def _out_body(p_ref, deg_ref, o_ref):
    o_ref[...] = (p_ref[0] + p_ref[1]) * _dis_of(deg_ref)[:, None]


_BE = 2000
_BN = 1024

_full = lambda shape: pl.BlockSpec(shape, lambda i: (0,) * len(shape))


def _m_tc(ea, we, be, ew, eb):
    return pl.pallas_call(
        _m_body,
        grid=(E // _BE,),
        in_specs=[
            pl.BlockSpec((_BE, ED), lambda i: (i, 0)),
            _full((ED, ED)), _full((ED,)), _full((D, ED)), _full((D,)),
        ],
        out_specs=pl.BlockSpec((_BE, D), lambda i: (i, 0)),
        out_shape=jax.ShapeDtypeStruct((E, D), jnp.bfloat16),
    )(ea, we, be, ew, eb)


def _h0_tc(x, lw, lb, degp):
    return pl.pallas_call(
        _h0_body,
        grid=(NP // _BN,),
        in_specs=[
            pl.BlockSpec((_BN, D), lambda i: (i, 0)),
            _full((D, D)), _full((D,)),
            pl.BlockSpec((NC, _BN), lambda i: (0, i)),
        ],
        out_specs=pl.BlockSpec((_BN, D), lambda i: (i, 0)),
        out_shape=jax.ShapeDtypeStruct((NP, D), F32),
    )(x, lw, lb, degp)


def _hmid_tc(part, degp, g, b, lw, lb):
    return pl.pallas_call(
        _hmid_body,
        grid=(NP // _BN,),
        in_specs=[
            pl.BlockSpec((NC, _BN, D), lambda i: (0, i, 0)),
            pl.BlockSpec((NC, _BN), lambda i: (0, i)),
            _full((D,)), _full((D,)), _full((D, D)), _full((D,)),
        ],
        out_specs=pl.BlockSpec((_BN, D), lambda i: (i, 0)),
        out_shape=jax.ShapeDtypeStruct((NP, D), F32),
    )(part, degp, g, b, lw, lb)


def _out_tc(part, degp):
    return pl.pallas_call(
        _out_body,
        grid=(NP // _BN,),
        in_specs=[
            pl.BlockSpec((NC, _BN, D), lambda i: (0, i, 0)),
            pl.BlockSpec((NC, _BN), lambda i: (0, i)),
        ],
        out_specs=pl.BlockSpec((_BN, D), lambda i: (i, 0)),
        out_shape=jax.ShapeDtypeStruct((NP, D), F32),
    )(part, degp)


def kernel(x, edge_index, edge_attr, edge_weight, edge_bias,
           lin_w0, lin_b0, et_w0, et_b0,
           lin_w1, lin_b1, et_w1, et_b1,
           lin_w2, lin_b2, et_w2, et_b2,
           ln_g0, ln_b0, ln_g1, ln_b1):
    row = edge_index[0]
    col = edge_index[1]
    x_pad = jnp.pad(x, ((0, NP - N), (0, 0)))
    degp = _deg_sc(col.reshape(NW, NB, EB)).reshape(NC, NP)
    h = _h0_tc(x_pad, lin_w0, lin_b0, degp)
    ews = [et_w0, et_w1, et_w2]
    ebs = [et_b0, et_b1, et_b2]
    lws = [lin_w1, lin_w2]
    lbs = [lin_b1, lin_b2]
    lgs = [ln_g0, ln_g1]
    lnb = [ln_b0, ln_b1]
    part = None
    for i in range(3):
        m = _m_tc(edge_attr, edge_weight, edge_bias,
                  ews[i][_PERM], ebs[i][_PERM])
        m32 = lax.bitcast_convert_type(
            m.reshape(E // EB, EB, D // 2, 2), jnp.int32)
        part = _mp_sc(h, m32, row, col)
        if i < 2:
            h = _hmid_tc(part, degp, lgs[i], lnb[i], lws[i], lbs[i])
    return _out_tc(part, degp)[:N]


# speedup vs baseline: 3.1474x; 3.1474x over previous
import functools

import numpy as np

import jax
import jax.numpy as jnp
from jax import lax
from jax.experimental import pallas as pl
from jax.experimental.pallas import tpu as pltpu
from jax.experimental.pallas import tpu_sc as plsc

N = 10000
E = 320000
D = 128
ED = 16
NP = 10240
NC = 2
NS = 16
NW = NC * NS
EPW = E // NW
EB = 80
NB = EPW // EB
RPT = NP // NS
ZR = 16
F32 = jnp.float32

_mesh = plsc.VectorSubcoreMesh(core_axis_name="c", subcore_axis_name="s")

_IDXL = np.array([32 * (j // 16) + (j % 16) for j in range(D // 2)],
                 dtype=np.int32)
_IDXH = _IDXL + 16


@functools.partial(
    pl.kernel,
    out_type=jax.ShapeDtypeStruct((NC, 1, NP), F32),
    mesh=_mesh,
    scratch_types=[
        pltpu.VMEM_SHARED((NP,), F32),
        pltpu.VMEM((NB, EB), jnp.int32),
        pltpu.VMEM((EB,), F32),
        pltpu.VMEM((RPT,), F32),
        pltpu.SemaphoreType.DMA,
    ],
)
def _deg_sc(col_hbm, deg_hbm, acc, colb, ones, zbuf, si):
    cid = lax.axis_index("c")
    sid = lax.axis_index("s")
    wid = sid * NC + cid
    pltpu.async_copy(col_hbm.at[wid], colb, si)
    for k in range(RPT // 16):
        zbuf[pl.ds(k * 16, 16)] = jnp.zeros((16,), F32)
    for k in range(EB // 16):
        ones[pl.ds(k * 16, 16)] = jnp.full((16,), 1.0, F32)
    pltpu.sync_copy(zbuf, acc.at[pl.ds(sid * RPT, RPT)])
    pltpu.make_async_copy(col_hbm.at[wid], colb, si).wait()
    plsc.subcore_barrier()

    def blk(i, _):
        pltpu.sync_copy(ones, acc.at[colb.at[i]], add=True)
        return 0

    lax.fori_loop(0, NB, blk, 0)
    plsc.subcore_barrier()
    s = pl.ds(sid * RPT, RPT)
    pltpu.sync_copy(acc.at[s], zbuf)
    pltpu.sync_copy(zbuf, deg_hbm.at[cid, 0, s])


@functools.partial(
    pl.kernel,
    out_type=jax.ShapeDtypeStruct((NC, NP, D), F32),
    mesh=_mesh,
    scratch_types=[
        pltpu.VMEM_SHARED((NP, D), F32),
        [pltpu.VMEM((EB,), jnp.int32)] * 2,
        [pltpu.VMEM((EB,), jnp.int32)] * 2,
        [pltpu.VMEM((EB, D), F32)] * 2,
        [pltpu.VMEM((EB, D // 2), jnp.int32)] * 2,
        pltpu.VMEM((ZR, D), F32),
        [pltpu.SemaphoreType.DMA] * 2,
        [pltpu.SemaphoreType.DMA] * 2,
        [pltpu.SemaphoreType.DMA] * 2,
    ],
)
def _mp_sc(h_hbm, m_hbm, row_hbm, col_hbm, out_hbm, acc, rowv, colv, rows, mv,
           zb, sg, sm, si):
    cid = lax.axis_index("c")
    sid = lax.axis_index("s")
    wid = sid * NC + cid
    base = wid * EPW

    def zrow(r, _):
        for c in range(D // 16):
            zb[r, pl.ds(c * 16, 16)] = jnp.zeros((16,), F32)
        return 0

    lax.fori_loop(0, ZR, zrow, 0)

    def zcopy(k, _):
        pltpu.sync_copy(zb, acc.at[pl.ds(sid * RPT + k * ZR, ZR)])
        return 0

    lax.fori_loop(0, RPT // ZR, zcopy, 0)
    plsc.subcore_barrier()

    def idx_copies(i, t):
        sl = pl.ds(base + i * EB, EB)
        return (pltpu.make_async_copy(row_hbm.at[sl], rowv[t], si[t]),
                pltpu.make_async_copy(col_hbm.at[sl], colv[t], si[t]))

    def g_copy(i, t):
        return pltpu.make_async_copy(h_hbm.at[rowv[t]], rows[t], sg[t])

    def m_copy(i, t):
        return pltpu.make_async_copy(m_hbm.at[wid * NB + i], mv[t], sm[t])

    for c in idx_copies(0, 0):
        c.start()
        c.wait()
    g_copy(0, 0).start()
    m_copy(0, 0).start()
    for c in idx_copies(1, 1):
        c.start()

    def slot(i, s, t):
        g_copy(i, s).wait()
        m_copy(i, s).wait()

        @pl.when(i + 1 < NB)
        def _():
            for c in idx_copies(i + 1, t):
                c.wait()
            g_copy(i + 1, t).start()
            m_copy(i + 1, t).start()

        def mul(b, _):
            for g in range(D // 32):
                w = mv[s][b, pl.ds(16 * g, 16)]
                lo = lax.bitcast_convert_type(w << 16, F32)
                hi = lax.bitcast_convert_type(w & jnp.int32(-65536), F32)
                sl = pl.ds(32 * g, 16)
                sh = pl.ds(32 * g + 16, 16)
                rows[s][b, sl] = rows[s][b, sl] * lo
                rows[s][b, sh] = rows[s][b, sh] * hi
            return 0

        lax.fori_loop(0, EB, mul, 0)
        pltpu.sync_copy(rows[s], acc.at[colv[s]], add=True)

        @pl.when(i + 2 < NB)
        def _():
            for c in idx_copies(i + 2, s):
                c.start()

    def ring(k, _):
        i0 = 2 * k
        slot(i0, 0, 1)

        @pl.when(i0 + 1 < NB)
        def _():
            slot(i0 + 1, 1, 0)

        return 0

    lax.fori_loop(0, (NB + 1) // 2, ring, 0)
    plsc.subcore_barrier()

    def wb(k, _):
        s = pl.ds(sid * RPT + k * ZR, ZR)
        pltpu.sync_copy(acc.at[s], zb)
        pltpu.sync_copy(zb, out_hbm.at[cid, s])
        return 0

    lax.fori_loop(0, RPT // ZR, wb, 0)


def _dis_of(deg_ref):
    deg = deg_ref[0, :] + deg_ref[1, :]
    return jnp.where(deg > 0, lax.rsqrt(deg), 0.0)


def _bf16_bits_rne(x):
    b = lax.bitcast_convert_type(x, jnp.uint32)
    return (b + jnp.uint32(0x7FFF) + ((b >> 16) & jnp.uint32(1))) >> 16


def _m_body(ea_ref, we_ref, be_ref, ewl_ref, ebl_ref, ewh_ref, ebh_ref,
            o_ref):
    ef = lax.dot_general(ea_ref[...], we_ref[...], (((1,), (0,)), ((), ())),
                         preferred_element_type=F32)
    ef = ef + be_ref[...][None, :]
    mlo = lax.dot_general(ef, ewl_ref[...], (((1,), (1,)), ((), ())),
                          preferred_element_type=F32) + ebl_ref[...][None, :]
    mhi = lax.dot_general(ef, ewh_ref[...], (((1,), (1,)), ((), ())),
                          preferred_element_type=F32) + ebh_ref[...][None, :]
    w = _bf16_bits_rne(mlo) | (_bf16_bits_rne(mhi) << 16)
    o_ref[...] = lax.bitcast_convert_type(w, jnp.int32)


def _h0_body(x_ref, lw_ref, lb_ref, deg_ref, o_ref):
    h = lax.dot_general(x_ref[...], lw_ref[...], (((1,), (1,)), ((), ())),
                        preferred_element_type=F32)
    h = h + lb_ref[...][None, :]
    o_ref[...] = h * _dis_of(deg_ref)[:, None]


def _hmid_body(p_ref, deg_ref, g_ref, b_ref, lw_ref, lb_ref, o_ref):
    dis = _dis_of(deg_ref)
    y = (p_ref[0] + p_ref[1]) * dis[:, None]
    y = jnp.maximum(y, 0.0)
    mu = jnp.mean(y, axis=-1, keepdims=True)
    var = jnp.mean((y - mu) ** 2, axis=-1, keepdims=True)
    z = (y - mu) / jnp.sqrt(var + 1e-5) * g_ref[...][None, :] + b_ref[...][None, :]
    h = lax.dot_general(z, lw_ref[...], (((1,), (1,)), ((), ())),
                        preferred_element_type=F32)
    h = h + lb_ref[...][None, :]
    o_ref[...] = h * dis[:, None]


def _out_body(p_ref, deg_ref, o_ref):
    o_ref[...] = (p_ref[0] + p_ref[1]) * _dis_of(deg_ref)[:, None]


_BE = 2000
_BN = 1024

_full = lambda shape: pl.BlockSpec(shape, lambda i: (0,) * len(shape))


def _m_tc(ea, we, be, ew, eb):
    return pl.pallas_call(
        _m_body,
        grid=(E // _BE,),
        in_specs=[
            pl.BlockSpec((_BE, ED), lambda i: (i, 0)),
            _full((ED, ED)), _full((ED,)),
            _full((D // 2, ED)), _full((D // 2,)),
            _full((D // 2, ED)), _full((D // 2,)),
        ],
        out_specs=pl.BlockSpec((_BE, D // 2), lambda i: (i, 0)),
        out_shape=jax.ShapeDtypeStruct((E, D // 2), jnp.int32),
    )(ea, we, be, ew[_IDXL], eb[_IDXL], ew[_IDXH], eb[_IDXH])


def _h0_tc(x, lw, lb, degp):
    return pl.pallas_call(
        _h0_body,
        grid=(NP // _BN,),
        in_specs=[
            pl.BlockSpec((_BN, D), lambda i: (i, 0)),
            _full((D, D)), _full((D,)),
            pl.BlockSpec((NC, _BN), lambda i: (0, i)),
        ],
        out_specs=pl.BlockSpec((_BN, D), lambda i: (i, 0)),
        out_shape=jax.ShapeDtypeStruct((NP, D), F32),
    )(x, lw, lb, degp)


def _hmid_tc(part, degp, g, b, lw, lb):
    return pl.pallas_call(
        _hmid_body,
        grid=(NP // _BN,),
        in_specs=[
            pl.BlockSpec((NC, _BN, D), lambda i: (0, i, 0)),
            pl.BlockSpec((NC, _BN), lambda i: (0, i)),
            _full((D,)), _full((D,)), _full((D, D)), _full((D,)),
        ],
        out_specs=pl.BlockSpec((_BN, D), lambda i: (i, 0)),
        out_shape=jax.ShapeDtypeStruct((NP, D), F32),
    )(part, degp, g, b, lw, lb)


def _out_tc(part, degp):
    return pl.pallas_call(
        _out_body,
        grid=(NP // _BN,),
        in_specs=[
            pl.BlockSpec((NC, _BN, D), lambda i: (0, i, 0)),
            pl.BlockSpec((NC, _BN), lambda i: (0, i)),
        ],
        out_specs=pl.BlockSpec((_BN, D), lambda i: (i, 0)),
        out_shape=jax.ShapeDtypeStruct((NP, D), F32),
    )(part, degp)


def kernel(x, edge_index, edge_attr, edge_weight, edge_bias,
           lin_w0, lin_b0, et_w0, et_b0,
           lin_w1, lin_b1, et_w1, et_b1,
           lin_w2, lin_b2, et_w2, et_b2,
           ln_g0, ln_b0, ln_g1, ln_b1):
    row = edge_index[0]
    col = edge_index[1]
    x_pad = jnp.pad(x, ((0, NP - N), (0, 0)))
    degp = _deg_sc(col.reshape(NW, NB, EB)).reshape(NC, NP)
    h = _h0_tc(x_pad, lin_w0, lin_b0, degp)
    ews = [et_w0, et_w1, et_w2]
    ebs = [et_b0, et_b1, et_b2]
    lws = [lin_w1, lin_w2]
    lbs = [lin_b1, lin_b2]
    lgs = [ln_g0, ln_g1]
    lnb = [ln_b0, ln_b1]
    part = None
    for i in range(3):
        m32 = _m_tc(edge_attr, edge_weight, edge_bias, ews[i], ebs[i])
        part = _mp_sc(h, m32.reshape(E // EB, EB, D // 2), row, col)
        if i < 2:
            h = _hmid_tc(part, degp, lgs[i], lnb[i], lws[i], lbs[i])
    return _out_tc(part, degp)[:N]


# speedup vs baseline: 3.1489x; 1.0005x over previous
import functools

import numpy as np

import jax
import jax.numpy as jnp
from jax import lax
from jax.experimental import pallas as pl
from jax.experimental.pallas import tpu as pltpu
from jax.experimental.pallas import tpu_sc as plsc

N = 10000
E = 320000
D = 128
ED = 16
NP = 10240
NC = 2
NS = 16
NW = NC * NS
EPW = E // NW
EB = 80
NB = EPW // EB
RPT = NP // NS
ZR = 16
F32 = jnp.float32

_mesh = plsc.VectorSubcoreMesh(core_axis_name="c", subcore_axis_name="s")

_IDXL = np.array([32 * (j // 16) + (j % 16) for j in range(D // 2)],
                 dtype=np.int32)
_IDXH = _IDXL + 16


@functools.partial(
    pl.kernel,
    out_type=jax.ShapeDtypeStruct((NC, 1, NP), F32),
    mesh=_mesh,
    scratch_types=[
        pltpu.VMEM_SHARED((NP,), F32),
        pltpu.VMEM((NB, EB), jnp.int32),
        pltpu.VMEM((EB,), F32),
        pltpu.VMEM((RPT,), F32),
        pltpu.SemaphoreType.DMA,
    ],
)
def _deg_sc(col_hbm, deg_hbm, acc, colb, ones, zbuf, si):
    cid = lax.axis_index("c")
    sid = lax.axis_index("s")
    wid = sid * NC + cid
    pltpu.async_copy(col_hbm.at[wid], colb, si)
    for k in range(RPT // 16):
        zbuf[pl.ds(k * 16, 16)] = jnp.zeros((16,), F32)
    for k in range(EB // 16):
        ones[pl.ds(k * 16, 16)] = jnp.full((16,), 1.0, F32)
    pltpu.sync_copy(zbuf, acc.at[pl.ds(sid * RPT, RPT)])
    pltpu.make_async_copy(col_hbm.at[wid], colb, si).wait()
    plsc.subcore_barrier()

    def blk(i, _):
        pltpu.sync_copy(ones, acc.at[colb.at[i]], add=True)
        return 0

    lax.fori_loop(0, NB, blk, 0)
    plsc.subcore_barrier()
    s = pl.ds(sid * RPT, RPT)
    pltpu.sync_copy(acc.at[s], zbuf)
    pltpu.sync_copy(zbuf, deg_hbm.at[cid, 0, s])


@functools.partial(
    pl.kernel,
    out_type=jax.ShapeDtypeStruct((NC, NP, D), F32),
    mesh=_mesh,
    scratch_types=[
        pltpu.VMEM_SHARED((NP, D), F32),
        [pltpu.VMEM((EB,), jnp.int32)] * 2,
        [pltpu.VMEM((EB,), jnp.int32)] * 2,
        [pltpu.VMEM((EB, D), F32)] * 2,
        [pltpu.VMEM((EB, D // 2), jnp.int32)] * 2,
        pltpu.VMEM((ZR, D), F32),
        [pltpu.SemaphoreType.DMA] * 2,
        [pltpu.SemaphoreType.DMA] * 2,
        [pltpu.SemaphoreType.DMA] * 2,
    ],
)
def _mp_sc(h_hbm, m_hbm, row_hbm, col_hbm, out_hbm, acc, rowv, colv, rows, mv,
           zb, sg, sm, si):
    cid = lax.axis_index("c")
    sid = lax.axis_index("s")
    wid = sid * NC + cid
    base = wid * EPW

    def zrow(r, _):
        for c in range(D // 16):
            zb[r, pl.ds(c * 16, 16)] = jnp.zeros((16,), F32)
        return 0

    lax.fori_loop(0, ZR, zrow, 0)

    def zcopy(k, _):
        pltpu.sync_copy(zb, acc.at[pl.ds(sid * RPT + k * ZR, ZR)])
        return 0

    lax.fori_loop(0, RPT // ZR, zcopy, 0)
    plsc.subcore_barrier()

    def idx_copies(i, t):
        sl = pl.ds(base + i * EB, EB)
        return (pltpu.make_async_copy(row_hbm.at[sl], rowv[t], si[t]),
                pltpu.make_async_copy(col_hbm.at[sl], colv[t], si[t]))

    def g_copy(t):
        return pltpu.make_async_copy(h_hbm.at[rowv[t]], rows[t], sg[t])

    def m_copy(i, t):
        return pltpu.make_async_copy(m_hbm.at[wid * NB + i], mv[t], sm[t])

    for c in idx_copies(0, 0):
        c.start()
        c.wait()
    g_copy(0).start()
    m_copy(0, 0).start()
    for c in idx_copies(1, 1):
        c.start()

    def slot(i, s, t):
        g_copy(s).wait()
        m_copy(i, s).wait()

        @pl.when(i + 1 < NB)
        def _():
            for c in idx_copies(i + 1, t):
                c.wait()
            g_copy(t).start()
            m_copy(i + 1, t).start()

        def mul(b, _):
            for g in range(D // 32):
                w = mv[s][b, pl.ds(16 * g, 16)]
                lo = lax.bitcast_convert_type(w << 16, F32)
                hi = lax.bitcast_convert_type(w & jnp.int32(-65536), F32)
                sl = pl.ds(32 * g, 16)
                sh = pl.ds(32 * g + 16, 16)
                rows[s][b, sl] = rows[s][b, sl] * lo
                rows[s][b, sh] = rows[s][b, sh] * hi
            return 0

        lax.fori_loop(0, EB, mul, 0)
        pltpu.sync_copy(rows[s], acc.at[colv[s]], add=True)

        @pl.when(i + 2 < NB)
        def _():
            for c in idx_copies(i + 2, s):
                c.start()

    def ring(k, _):
        i0 = 2 * k
        slot(i0, 0, 1)

        @pl.when(i0 + 1 < NB)
        def _():
            slot(i0 + 1, 1, 0)

        return 0

    lax.fori_loop(0, (NB + 1) // 2, ring, 0)
    plsc.subcore_barrier()

    def wb(k, _):
        s = pl.ds(sid * RPT + k * ZR, ZR)
        pltpu.sync_copy(acc.at[s], zb)
        pltpu.sync_copy(zb, out_hbm.at[cid, s])
        return 0

    lax.fori_loop(0, RPT // ZR, wb, 0)


def _dis_of(deg_ref):
    deg = deg_ref[0, :] + deg_ref[1, :]
    return jnp.where(deg > 0, lax.rsqrt(deg), 0.0)


def _bf16_bits_rne(x):
    b = lax.bitcast_convert_type(x, jnp.uint32)
    return (b + jnp.uint32(0x7FFF) + ((b >> 16) & jnp.uint32(1))) >> 16


def _m_body(ea_ref, we_ref, be_ref, ewl_ref, ebl_ref, ewh_ref, ebh_ref,
            o_ref):
    ef = lax.dot_general(ea_ref[...], we_ref[...], (((1,), (0,)), ((), ())),
                         preferred_element_type=F32)
    ef = ef + be_ref[...][None, :]
    mlo = lax.dot_general(ef, ewl_ref[...], (((1,), (1,)), ((), ())),
                          preferred_element_type=F32) + ebl_ref[...][None, :]
    mhi = lax.dot_general(ef, ewh_ref[...], (((1,), (1,)), ((), ())),
                          preferred_element_type=F32) + ebh_ref[...][None, :]
    w = _bf16_bits_rne(mlo) | (_bf16_bits_rne(mhi) << 16)
    o_ref[...] = lax.bitcast_convert_type(w, jnp.int32)


def _h0_body(x_ref, lw_ref, lb_ref, deg_ref, o_ref):
    h = lax.dot_general(x_ref[...], lw_ref[...], (((1,), (1,)), ((), ())),
                        preferred_element_type=F32)
    h = h + lb_ref[...][None, :]
    o_ref[...] = h * _dis_of(deg_ref)[:, None]


def _hmid_body(p_ref, deg_ref, g_ref, b_ref, lw_ref, lb_ref, o_ref):
    dis = _dis_of(deg_ref)
    y = (p_ref[0] + p_ref[1]) * dis[:, None]
    y = jnp.maximum(y, 0.0)
    mu = jnp.mean(y, axis=-1, keepdims=True)
    var = jnp.mean((y - mu) ** 2, axis=-1, keepdims=True)
    z = (y - mu) / jnp.sqrt(var + 1e-5) * g_ref[...][None, :] + b_ref[...][None, :]
    h = lax.dot_general(z, lw_ref[...], (((1,), (1,)), ((), ())),
                        preferred_element_type=F32)
    h = h + lb_ref[...][None, :]
    o_ref[...] = h * dis[:, None]


def _out_body(p_ref, deg_ref, o_ref):
    o_ref[...] = (p_ref[0] + p_ref[1]) * _dis_of(deg_ref)[:, None]


_BE = 2000
_BN = 1024

_full = lambda shape: pl.BlockSpec(shape, lambda i: (0,) * len(shape))


def _m_tc(ea, we, be, ew, eb):
    return pl.pallas_call(
        _m_body,
        grid=(E // _BE,),
        in_specs=[
            pl.BlockSpec((_BE, ED), lambda i: (i, 0)),
            _full((ED, ED)), _full((ED,)),
            _full((D // 2, ED)), _full((D // 2,)),
            _full((D // 2, ED)), _full((D // 2,)),
        ],
        out_specs=pl.BlockSpec((_BE, D // 2), lambda i: (i, 0)),
        out_shape=jax.ShapeDtypeStruct((E, D // 2), jnp.int32),
    )(ea, we, be, ew[_IDXL], eb[_IDXL], ew[_IDXH], eb[_IDXH])


def _h0_tc(x, lw, lb, degp):
    return pl.pallas_call(
        _h0_body,
        grid=(NP // _BN,),
        in_specs=[
            pl.BlockSpec((_BN, D), lambda i: (i, 0)),
            _full((D, D)), _full((D,)),
            pl.BlockSpec((NC, _BN), lambda i: (0, i)),
        ],
        out_specs=pl.BlockSpec((_BN, D), lambda i: (i, 0)),
        out_shape=jax.ShapeDtypeStruct((NP, D), F32),
    )(x, lw, lb, degp)


def _hmid_tc(part, degp, g, b, lw, lb):
    return pl.pallas_call(
        _hmid_body,
        grid=(NP // _BN,),
        in_specs=[
            pl.BlockSpec((NC, _BN, D), lambda i: (0, i, 0)),
            pl.BlockSpec((NC, _BN), lambda i: (0, i)),
            _full((D,)), _full((D,)), _full((D, D)), _full((D,)),
        ],
        out_specs=pl.BlockSpec((_BN, D), lambda i: (i, 0)),
        out_shape=jax.ShapeDtypeStruct((NP, D), F32),
    )(part, degp, g, b, lw, lb)


def _out_tc(part, degp):
    return pl.pallas_call(
        _out_body,
        grid=(NP // _BN,),
        in_specs=[
            pl.BlockSpec((NC, _BN, D), lambda i: (0, i, 0)),
            pl.BlockSpec((NC, _BN), lambda i: (0, i)),
        ],
        out_specs=pl.BlockSpec((_BN, D), lambda i: (i, 0)),
        out_shape=jax.ShapeDtypeStruct((NP, D), F32),
    )(part, degp)


def kernel(x, edge_index, edge_attr, edge_weight, edge_bias,
           lin_w0, lin_b0, et_w0, et_b0,
           lin_w1, lin_b1, et_w1, et_b1,
           lin_w2, lin_b2, et_w2, et_b2,
           ln_g0, ln_b0, ln_g1, ln_b1):
    row = edge_index[0]
    col = edge_index[1]
    x_pad = jnp.pad(x, ((0, NP - N), (0, 0)))
    degp = _deg_sc(col.reshape(NW, NB, EB)).reshape(NC, NP)
    h = _h0_tc(x_pad, lin_w0, lin_b0, degp)
    ews = [et_w0, et_w1, et_w2]
    ebs = [et_b0, et_b1, et_b2]
    lws = [lin_w1, lin_w2]
    lbs = [lin_b1, lin_b2]
    lgs = [ln_g0, ln_g1]
    lnb = [ln_b0, ln_b1]
    part = None
    for i in range(3):
        m32 = _m_tc(edge_attr, edge_weight, edge_bias, ews[i], ebs[i])
        part = _mp_sc(h, m32.reshape(E // EB, EB, D // 2), row, col)
        if i < 2:
            h = _hmid_tc(part, degp, lgs[i], lnb[i], lws[i], lbs[i])
    return _out_tc(part, degp)[:N]
